# Initial kernel scaffold; baseline (speedup 1.0000x reference)
import jax
import jax.numpy as jnp
from jax.experimental import pallas as pl


def kernel(h_src, h_dst, edge_feat, edge_index, forward_mask, reverse_mask, WI, bI, WO, bO, WR, bR):
    raise NotImplementedError("write your pallas kernel here")


# speedup vs baseline: 4.3144x; 4.3144x over previous
import functools

import jax
import jax.numpy as jnp
from jax import lax
from jax.experimental import pallas as pl
from jax.experimental.pallas import tpu as pltpu
from jax.experimental.pallas import tpu_sc as plsc

N_SRC = 10000
N_DST = 10000
E = 320000
D = 128
DH = 64
NC = 2
NS = 16
CHUNK = 80
EPT = E // NS
NCH = EPT // CHUNK
TROWS = 20480
RPT = TROWS // NS
W = 25
NW = NCH // W


def _sc_seg_body(hcat, ef, src3, dst3, fwd3, out_s, out_c,
                 dstv, fwdv, srcv, h_v, ef_v, val_v, ones_v, zv,
                 table, cnt, sem_h, sem_e):
    c = lax.axis_index("c")
    s = lax.axis_index("s")
    ebase = s * EPT

    for k in range(CHUNK // 16):
        ones_v[pl.ds(16 * k, 16)] = jnp.full((16,), 1.0, jnp.float32)
        zv[pl.ds(16 * k, 16)] = jnp.zeros((16,), jnp.float32)

    def _zero_val(r, carry):
        for k in range(DH // 16):
            val_v[r, pl.ds(16 * k, 16)] = jnp.zeros((16,), jnp.float32)
        return carry
    lax.fori_loop(0, CHUNK, _zero_val, 0)

    zrow = s * RPT
    for z in range(RPT // CHUNK):
        pltpu.sync_copy(val_v, table.at[pl.ds(zrow + z * CHUNK, CHUNK)])
        pltpu.sync_copy(zv, cnt.at[pl.ds(zrow + z * CHUNK, CHUNK)])
    plsc.subcore_barrier()

    def _window(w, carry):
        pltpu.sync_copy(dst3.at[s, pl.ds(w * W, W)], dstv)
        pltpu.sync_copy(fwd3.at[s, pl.ds(w * W, W)], fwdv)
        pltpu.sync_copy(src3.at[s, pl.ds(w * W, W)], srcv)

        def _mkidx(r, carry2):
            for k in range(CHUNK // 16):
                sl = pl.ds(16 * k, 16)
                dstv[r, sl] = dstv[r, sl] + fwdv[r, sl] * N_DST
                srcv[r, sl] = srcv[r, sl] + c * N_SRC
            return carry2
        lax.fori_loop(0, W, _mkidx, 0)

        def _step(j, carry2):
            cp_h = pltpu.async_copy(hcat.at[srcv.at[j]], h_v, sem_h)
            cp_e = pltpu.async_copy(
                ef.at[pl.ds(ebase + (w * W + j) * CHUNK, CHUNK),
                      pl.ds(c * DH, DH)],
                ef_v, sem_e)
            cp_h.wait()
            cp_e.wait()

            def _diff(r, carry3):
                for k in range(DH // 16):
                    sl = pl.ds(16 * k, 16)
                    val_v[r, sl] = h_v[r, sl] - ef_v[r, sl]
                return carry3
            lax.fori_loop(0, CHUNK, _diff, 0)

            pltpu.sync_copy(val_v, table.at[dstv.at[j]], add=True)
            pltpu.sync_copy(ones_v, cnt.at[dstv.at[j]], add=True)
            return carry2
        lax.fori_loop(0, W, _step, 0)
        return carry
    lax.fori_loop(0, NW, _window, 0)

    plsc.subcore_barrier()
    pltpu.sync_copy(table.at[pl.ds(zrow, RPT)], out_s.at[c, pl.ds(zrow, RPT)])
    pltpu.sync_copy(cnt.at[pl.ds(zrow, RPT)], out_c.at[c, pl.ds(zrow, RPT)])


def _sc_seg(hcat, ef, src3, dst3, fwd3):
    mesh = plsc.VectorSubcoreMesh(core_axis_name="c", subcore_axis_name="s")
    f = functools.partial(
        pl.kernel,
        mesh=mesh,
        compiler_params=pltpu.CompilerParams(use_tc_tiling_on_sc=False),
        out_type=[
            jax.ShapeDtypeStruct((NC, TROWS, DH), jnp.float32),
            jax.ShapeDtypeStruct((NC, TROWS), jnp.float32),
        ],
        scratch_types=[
            pltpu.VMEM((W, CHUNK), jnp.int32),
            pltpu.VMEM((W, CHUNK), jnp.int32),
            pltpu.VMEM((W, CHUNK), jnp.int32),
            pltpu.VMEM((CHUNK, DH), jnp.float32),
            pltpu.VMEM((CHUNK, DH), jnp.float32),
            pltpu.VMEM((CHUNK, DH), jnp.float32),
            pltpu.VMEM((CHUNK,), jnp.float32),
            pltpu.VMEM((CHUNK,), jnp.float32),
            pltpu.VMEM_SHARED((TROWS, DH), jnp.float32),
            pltpu.VMEM_SHARED((TROWS,), jnp.float32),
            pltpu.SemaphoreType.DMA,
            pltpu.SemaphoreType.DMA,
        ],
    )(_sc_seg_body)
    return f(hcat, ef, src3, dst3, fwd3)


def _n_assemble_body(sf, sr, cf, cr, wit, wot, bi, bo, o):
    acc = jnp.dot(sf[0], wit[:DH, :], preferred_element_type=jnp.float32)
    acc = acc + jnp.dot(sf[1], wit[DH:, :], preferred_element_type=jnp.float32)
    acc = acc + jnp.dot(sr[0], wot[:DH, :], preferred_element_type=jnp.float32)
    acc = acc + jnp.dot(sr[1], wot[DH:, :], preferred_element_type=jnp.float32)
    o[...] = acc + cf[0] * bi[...] + cr[0] * bo[...]


def _n_assemble(S, C3, wit, wot, bi, bo, interpret=False):
    R = 1000
    nfb = N_DST // R
    grid = (N_DST // R,)
    return pl.pallas_call(
        _n_assemble_body,
        grid=grid,
        in_specs=[
            pl.BlockSpec((NC, R, DH), lambda i: (0, i + nfb, 0)),
            pl.BlockSpec((NC, R, DH), lambda i: (0, i, 0)),
            pl.BlockSpec((NC, R, 1), lambda i: (0, i + nfb, 0)),
            pl.BlockSpec((NC, R, 1), lambda i: (0, i, 0)),
            pl.BlockSpec((D, D), lambda i: (0, 0)),
            pl.BlockSpec((D, D), lambda i: (0, 0)),
            pl.BlockSpec((1, D), lambda i: (0, 0)),
            pl.BlockSpec((1, D), lambda i: (0, 0)),
        ],
        out_specs=pl.BlockSpec((R, D), lambda i: (i, 0)),
        out_shape=jax.ShapeDtypeStruct((N_DST, D), jnp.float32),
        interpret=interpret,
    )(S, S, C3, C3, wit, wot, bi, bo)


def _e_out_body(x, w, b, o):
    o[...] = jnp.dot(x[...], w[...], preferred_element_type=jnp.float32) + b[...]


def _e_out(ef, wrt, br, interpret=False):
    BE = 2000
    return pl.pallas_call(
        _e_out_body,
        grid=(E // BE,),
        in_specs=[
            pl.BlockSpec((BE, D), lambda i: (i, 0)),
            pl.BlockSpec((D, D), lambda i: (0, 0)),
            pl.BlockSpec((1, D), lambda i: (0, 0)),
        ],
        out_specs=pl.BlockSpec((BE, D), lambda i: (i, 0)),
        out_shape=jax.ShapeDtypeStruct((E, D), jnp.float32),
        interpret=interpret,
    )(ef, wrt, br)


def kernel(h_src, h_dst, edge_feat, edge_index, forward_mask, reverse_mask,
           WI, bI, WO, bO, WR, bR):
    src = edge_index[0]
    dst = edge_index[1]
    fwd = forward_mask.astype(jnp.int32)
    hcat = jnp.concatenate([h_src[:, :DH], h_src[:, DH:]], axis=0)
    src3 = src.reshape(NS, NCH, CHUNK)
    dst3 = dst.reshape(NS, NCH, CHUNK)
    fwd3 = fwd.reshape(NS, NCH, CHUNK)

    S, C = _sc_seg(hcat, edge_feat, src3, dst3, fwd3)
    C3 = C.reshape(NC, TROWS, 1)
    n_out = _n_assemble(S, C3, WI.T, WO.T,
                        bI.reshape(1, D), bO.reshape(1, D))
    e_out = _e_out(edge_feat, WR.T, bR.reshape(1, D))
    return (n_out, e_out)


# speedup vs baseline: 5.0342x; 1.1668x over previous
import functools

import jax
import jax.numpy as jnp
from jax import lax
from jax.experimental import pallas as pl
from jax.experimental.pallas import tpu as pltpu
from jax.experimental.pallas import tpu_sc as plsc

N_SRC = 10000
N_DST = 10000
E = 320000
D = 128
DH = 64
NC = 2
NS = 16
CHUNK = 80
EPT = E // NS
NCH = EPT // CHUNK
TROWS = 20480
RPT = TROWS // NS
W = 50
NW = NCH // W


def _sc_seg_body(hcat, ef, src3, dst3, fwd3, out_s, out_c,
                 dstv, fwdv, srcv, h_0, ef_0, val_0, h_1, ef_1, val_1,
                 ones_v, zv, table, cnt, sem_h0, sem_e0, sem_h1, sem_e1):
    c = lax.axis_index("c")
    s = lax.axis_index("s")
    ebase = s * EPT

    for k in range(CHUNK // 16):
        ones_v[pl.ds(16 * k, 16)] = jnp.full((16,), 1.0, jnp.float32)
        zv[pl.ds(16 * k, 16)] = jnp.zeros((16,), jnp.float32)

    def _zero_val(r, carry):
        for k in range(DH // 16):
            val_0[r, pl.ds(16 * k, 16)] = jnp.zeros((16,), jnp.float32)
        return carry
    lax.fori_loop(0, CHUNK, _zero_val, 0)

    zrow = s * RPT
    for z in range(RPT // CHUNK):
        pltpu.sync_copy(val_0, table.at[pl.ds(zrow + z * CHUNK, CHUNK)])
        pltpu.sync_copy(zv, cnt.at[pl.ds(zrow + z * CHUNK, CHUNK)])
    plsc.subcore_barrier()

    def _window(w, carry):
        pltpu.sync_copy(dst3.at[s, pl.ds(w * W, W)], dstv)
        pltpu.sync_copy(fwd3.at[s, pl.ds(w * W, W)], fwdv)
        pltpu.sync_copy(src3.at[s, pl.ds(w * W, W)], srcv)

        def _mkidx(r, carry2):
            for k in range(CHUNK // 16):
                sl = pl.ds(16 * k, 16)
                dstv[r, sl] = dstv[r, sl] + fwdv[r, sl] * N_DST
                srcv[r, sl] = srcv[r, sl] + c * N_SRC
            return carry2
        lax.fori_loop(0, W, _mkidx, 0)

        def _fetch(j, h_b, ef_b, sem_hb, sem_eb):
            cp_h = pltpu.async_copy(hcat.at[srcv.at[j]], h_b, sem_hb)
            cp_e = pltpu.async_copy(
                ef.at[pl.ds(ebase + (w * W + j) * CHUNK, CHUNK),
                      pl.ds(c * DH, DH)],
                ef_b, sem_eb)
            return cp_h, cp_e

        def _consume(j, cp_h, cp_e, h_b, ef_b, val_b):
            cp_h.wait()
            cp_e.wait()

            def _diff(r, carry3):
                for k in range(DH // 16):
                    sl = pl.ds(16 * k, 16)
                    val_b[r, sl] = h_b[r, sl] - ef_b[r, sl]
                return carry3
            lax.fori_loop(0, CHUNK, _diff, 0)

            pltpu.sync_copy(val_b, table.at[dstv.at[j]], add=True)
            pltpu.sync_copy(ones_v, cnt.at[dstv.at[j]], add=True)

        def _pair(q, carry2):
            j0 = 2 * q
            j1 = 2 * q + 1
            cps0 = _fetch(j0, h_0, ef_0, sem_h0, sem_e0)
            cps1 = _fetch(j1, h_1, ef_1, sem_h1, sem_e1)
            _consume(j0, *cps0, h_0, ef_0, val_0)
            _consume(j1, *cps1, h_1, ef_1, val_1)
            return carry2
        lax.fori_loop(0, W // 2, _pair, 0)
        return carry
    lax.fori_loop(0, NW, _window, 0)

    plsc.subcore_barrier()
    pltpu.sync_copy(table.at[pl.ds(zrow, RPT)], out_s.at[c, pl.ds(zrow, RPT)])
    pltpu.sync_copy(cnt.at[pl.ds(zrow, RPT)], out_c.at[c, pl.ds(zrow, RPT)])


def _sc_seg(hcat, ef, src3, dst3, fwd3):
    mesh = plsc.VectorSubcoreMesh(core_axis_name="c", subcore_axis_name="s")
    f = functools.partial(
        pl.kernel,
        mesh=mesh,
        compiler_params=pltpu.CompilerParams(use_tc_tiling_on_sc=False),
        out_type=[
            jax.ShapeDtypeStruct((NC, TROWS, DH), jnp.float32),
            jax.ShapeDtypeStruct((NC, TROWS), jnp.float32),
        ],
        scratch_types=[
            pltpu.VMEM((W, CHUNK), jnp.int32),
            pltpu.VMEM((W, CHUNK), jnp.int32),
            pltpu.VMEM((W, CHUNK), jnp.int32),
            pltpu.VMEM((CHUNK, DH), jnp.float32),
            pltpu.VMEM((CHUNK, DH), jnp.float32),
            pltpu.VMEM((CHUNK, DH), jnp.float32),
            pltpu.VMEM((CHUNK, DH), jnp.float32),
            pltpu.VMEM((CHUNK, DH), jnp.float32),
            pltpu.VMEM((CHUNK, DH), jnp.float32),
            pltpu.VMEM((CHUNK,), jnp.float32),
            pltpu.VMEM((CHUNK,), jnp.float32),
            pltpu.VMEM_SHARED((TROWS, DH), jnp.float32),
            pltpu.VMEM_SHARED((TROWS,), jnp.float32),
            pltpu.SemaphoreType.DMA,
            pltpu.SemaphoreType.DMA,
            pltpu.SemaphoreType.DMA,
            pltpu.SemaphoreType.DMA,
        ],
    )(_sc_seg_body)
    return f(hcat, ef, src3, dst3, fwd3)


def _n_assemble_body(sf, sr, cf, cr, wit, wot, bi, bo, o):
    acc = jnp.dot(sf[0], wit[:DH, :], preferred_element_type=jnp.float32)
    acc = acc + jnp.dot(sf[1], wit[DH:, :], preferred_element_type=jnp.float32)
    acc = acc + jnp.dot(sr[0], wot[:DH, :], preferred_element_type=jnp.float32)
    acc = acc + jnp.dot(sr[1], wot[DH:, :], preferred_element_type=jnp.float32)
    o[...] = acc + cf[0] * bi[...] + cr[0] * bo[...]


def _n_assemble(S, C3, wit, wot, bi, bo, interpret=False):
    R = 1000
    nfb = N_DST // R
    grid = (N_DST // R,)
    return pl.pallas_call(
        _n_assemble_body,
        grid=grid,
        in_specs=[
            pl.BlockSpec((NC, R, DH), lambda i: (0, i + nfb, 0)),
            pl.BlockSpec((NC, R, DH), lambda i: (0, i, 0)),
            pl.BlockSpec((NC, R, 1), lambda i: (0, i + nfb, 0)),
            pl.BlockSpec((NC, R, 1), lambda i: (0, i, 0)),
            pl.BlockSpec((D, D), lambda i: (0, 0)),
            pl.BlockSpec((D, D), lambda i: (0, 0)),
            pl.BlockSpec((1, D), lambda i: (0, 0)),
            pl.BlockSpec((1, D), lambda i: (0, 0)),
        ],
        out_specs=pl.BlockSpec((R, D), lambda i: (i, 0)),
        out_shape=jax.ShapeDtypeStruct((N_DST, D), jnp.float32),
        interpret=interpret,
    )(S, S, C3, C3, wit, wot, bi, bo)


def _e_out_body(x, w, b, o):
    o[...] = jnp.dot(x[...], w[...], preferred_element_type=jnp.float32) + b[...]


def _e_out(ef, wrt, br, interpret=False):
    BE = 2000
    return pl.pallas_call(
        _e_out_body,
        grid=(E // BE,),
        in_specs=[
            pl.BlockSpec((BE, D), lambda i: (i, 0)),
            pl.BlockSpec((D, D), lambda i: (0, 0)),
            pl.BlockSpec((1, D), lambda i: (0, 0)),
        ],
        out_specs=pl.BlockSpec((BE, D), lambda i: (i, 0)),
        out_shape=jax.ShapeDtypeStruct((E, D), jnp.float32),
        interpret=interpret,
    )(ef, wrt, br)


def kernel(h_src, h_dst, edge_feat, edge_index, forward_mask, reverse_mask,
           WI, bI, WO, bO, WR, bR):
    src = edge_index[0]
    dst = edge_index[1]
    fwd = forward_mask.astype(jnp.int32)
    hcat = jnp.concatenate([h_src[:, :DH], h_src[:, DH:]], axis=0)
    src3 = src.reshape(NS, NCH, CHUNK)
    dst3 = dst.reshape(NS, NCH, CHUNK)
    fwd3 = fwd.reshape(NS, NCH, CHUNK)

    S, C = _sc_seg(hcat, edge_feat, src3, dst3, fwd3)
    C3 = C.reshape(NC, TROWS, 1)
    n_out = _n_assemble(S, C3, WI.T, WO.T,
                        bI.reshape(1, D), bO.reshape(1, D))
    e_out = _e_out(edge_feat, WR.T, bR.reshape(1, D))
    return (n_out, e_out)


# speedup vs baseline: 6.9061x; 1.3718x over previous
import functools

import jax
import jax.numpy as jnp
from jax import lax
from jax.experimental import pallas as pl
from jax.experimental.pallas import tpu as pltpu
from jax.experimental.pallas import tpu_sc as plsc

N_SRC = 10000
N_DST = 10000
E = 320000
D = 128
DH = 64
NC = 2
NS = 16
CHUNK = 80
EPT = E // NS
NCH = EPT // CHUNK
TROWS = 20480
RPT = TROWS // NS
W = 50
NW = NCH // W


def _sc_seg_body(hcat, ef, src3, dst3, fwd3, out_s, out_c,
                 dstv, fwdv, srcv, h_0, ef_0, val_0, h_1, ef_1, val_1,
                 ones_v, zv, table, cnt, sem_h0, sem_e0, sem_h1, sem_e1,
                 sem_s0, sem_c0, sem_s1, sem_c1):
    c = lax.axis_index("c")
    s = lax.axis_index("s")
    ebase = s * EPT

    for k in range(CHUNK // 16):
        ones_v[pl.ds(16 * k, 16)] = jnp.full((16,), 1.0, jnp.float32)
        zv[pl.ds(16 * k, 16)] = jnp.zeros((16,), jnp.float32)

    def _zero_val(r, carry):
        for k in range(DH // 16):
            val_0[r, pl.ds(16 * k, 16)] = jnp.zeros((16,), jnp.float32)
        return carry
    lax.fori_loop(0, CHUNK, _zero_val, 0)

    zrow = s * RPT
    for z in range(RPT // CHUNK):
        pltpu.sync_copy(val_0, table.at[pl.ds(zrow + z * CHUNK, CHUNK)])
        pltpu.sync_copy(zv, cnt.at[pl.ds(zrow + z * CHUNK, CHUNK)])
    plsc.subcore_barrier()

    def _window(w, carry):
        @pl.when(w > 0)
        def _():
            pltpu.make_async_copy(hcat.at[pl.ds(0, CHUNK)], val_0,
                                  sem_s0).wait()
            pltpu.make_async_copy(hcat.at[pl.ds(0, CHUNK)], val_1,
                                  sem_s1).wait()
            pltpu.make_async_copy(out_c.at[c, pl.ds(0, CHUNK)], zv,
                                  sem_c0).wait()
            pltpu.make_async_copy(out_c.at[c, pl.ds(0, CHUNK)], zv,
                                  sem_c1).wait()
        pltpu.sync_copy(dst3.at[s, pl.ds(w * W, W)], dstv)
        pltpu.sync_copy(fwd3.at[s, pl.ds(w * W, W)], fwdv)
        pltpu.sync_copy(src3.at[s, pl.ds(w * W, W)], srcv)

        def _mkidx(r, carry2):
            for k in range(CHUNK // 16):
                sl = pl.ds(16 * k, 16)
                dstv[r, sl] = dstv[r, sl] + fwdv[r, sl] * N_DST
                srcv[r, sl] = srcv[r, sl] + c * N_SRC
            return carry2
        lax.fori_loop(0, W, _mkidx, 0)

        def _fetch(j, h_b, ef_b, sem_hb, sem_eb):
            cp_h = pltpu.async_copy(hcat.at[srcv.at[j]], h_b, sem_hb)
            cp_e = pltpu.async_copy(
                ef.at[pl.ds(ebase + (w * W + j) * CHUNK, CHUNK),
                      pl.ds(c * DH, DH)],
                ef_b, sem_eb)
            return cp_h, cp_e

        def _drain_scat(val_b, sem_sb, sem_cb):
            pltpu.make_async_copy(hcat.at[pl.ds(0, CHUNK)], val_b,
                                  sem_sb).wait()
            pltpu.make_async_copy(out_c.at[c, pl.ds(0, CHUNK)], zv,
                                  sem_cb).wait()

        def _consume(j, q, cp_h, cp_e, h_b, ef_b, val_b, sem_sb, sem_cb):
            cp_h.wait()
            cp_e.wait()

            @pl.when(q > 0)
            def _():
                _drain_scat(val_b, sem_sb, sem_cb)

            @functools.partial(plsc.parallel_loop, 0, CHUNK, unroll=4)
            def _diff(r):
                for k in range(DH // 16):
                    sl = pl.ds(16 * k, 16)
                    val_b[r, sl] = h_b[r, sl] - ef_b[r, sl]

            pltpu.async_copy(val_b, table.at[dstv.at[j]], sem_sb, add=True)
            pltpu.async_copy(ones_v, cnt.at[dstv.at[j]], sem_cb, add=True)

        def _pair(q, carry2):
            j0 = 2 * q
            j1 = 2 * q + 1
            cps0 = _fetch(j0, h_0, ef_0, sem_h0, sem_e0)
            cps1 = _fetch(j1, h_1, ef_1, sem_h1, sem_e1)
            _consume(j0, q, *cps0, h_0, ef_0, val_0, sem_s0, sem_c0)
            _consume(j1, q, *cps1, h_1, ef_1, val_1, sem_s1, sem_c1)
            return carry2
        lax.fori_loop(0, W // 2, _pair, 0)
        return carry
    lax.fori_loop(0, NW, _window, 0)
    _final0 = pltpu.make_async_copy(hcat.at[pl.ds(0, CHUNK)], val_0, sem_s0)
    _final0.wait()
    pltpu.make_async_copy(hcat.at[pl.ds(0, CHUNK)], val_1, sem_s1).wait()
    pltpu.make_async_copy(out_c.at[c, pl.ds(0, CHUNK)], zv, sem_c0).wait()
    pltpu.make_async_copy(out_c.at[c, pl.ds(0, CHUNK)], zv, sem_c1).wait()

    plsc.subcore_barrier()
    pltpu.sync_copy(table.at[pl.ds(zrow, RPT)], out_s.at[c, pl.ds(zrow, RPT)])
    pltpu.sync_copy(cnt.at[pl.ds(zrow, RPT)], out_c.at[c, pl.ds(zrow, RPT)])


def _sc_seg(hcat, ef, src3, dst3, fwd3):
    mesh = plsc.VectorSubcoreMesh(core_axis_name="c", subcore_axis_name="s")
    f = functools.partial(
        pl.kernel,
        mesh=mesh,
        compiler_params=pltpu.CompilerParams(use_tc_tiling_on_sc=False),
        out_type=[
            jax.ShapeDtypeStruct((NC, TROWS, DH), jnp.float32),
            jax.ShapeDtypeStruct((NC, TROWS), jnp.float32),
        ],
        scratch_types=[
            pltpu.VMEM((W, CHUNK), jnp.int32),
            pltpu.VMEM((W, CHUNK), jnp.int32),
            pltpu.VMEM((W, CHUNK), jnp.int32),
            pltpu.VMEM((CHUNK, DH), jnp.float32),
            pltpu.VMEM((CHUNK, DH), jnp.float32),
            pltpu.VMEM((CHUNK, DH), jnp.float32),
            pltpu.VMEM((CHUNK, DH), jnp.float32),
            pltpu.VMEM((CHUNK, DH), jnp.float32),
            pltpu.VMEM((CHUNK, DH), jnp.float32),
            pltpu.VMEM((CHUNK,), jnp.float32),
            pltpu.VMEM((CHUNK,), jnp.float32),
            pltpu.VMEM_SHARED((TROWS, DH), jnp.float32),
            pltpu.VMEM_SHARED((TROWS,), jnp.float32),
            pltpu.SemaphoreType.DMA,
            pltpu.SemaphoreType.DMA,
            pltpu.SemaphoreType.DMA,
            pltpu.SemaphoreType.DMA,
            pltpu.SemaphoreType.DMA,
            pltpu.SemaphoreType.DMA,
            pltpu.SemaphoreType.DMA,
            pltpu.SemaphoreType.DMA,
        ],
    )(_sc_seg_body)
    return f(hcat, ef, src3, dst3, fwd3)


def _n_assemble_body(sf, sr, cf, cr, wit, wot, bi, bo, o):
    acc = jnp.dot(sf[0], wit[:DH, :], preferred_element_type=jnp.float32)
    acc = acc + jnp.dot(sf[1], wit[DH:, :], preferred_element_type=jnp.float32)
    acc = acc + jnp.dot(sr[0], wot[:DH, :], preferred_element_type=jnp.float32)
    acc = acc + jnp.dot(sr[1], wot[DH:, :], preferred_element_type=jnp.float32)
    o[...] = acc + cf[0] * bi[...] + cr[0] * bo[...]


def _n_assemble(S, C3, wit, wot, bi, bo, interpret=False):
    R = 1000
    nfb = N_DST // R
    grid = (N_DST // R,)
    return pl.pallas_call(
        _n_assemble_body,
        grid=grid,
        in_specs=[
            pl.BlockSpec((NC, R, DH), lambda i: (0, i + nfb, 0)),
            pl.BlockSpec((NC, R, DH), lambda i: (0, i, 0)),
            pl.BlockSpec((NC, R, 1), lambda i: (0, i + nfb, 0)),
            pl.BlockSpec((NC, R, 1), lambda i: (0, i, 0)),
            pl.BlockSpec((D, D), lambda i: (0, 0)),
            pl.BlockSpec((D, D), lambda i: (0, 0)),
            pl.BlockSpec((1, D), lambda i: (0, 0)),
            pl.BlockSpec((1, D), lambda i: (0, 0)),
        ],
        out_specs=pl.BlockSpec((R, D), lambda i: (i, 0)),
        out_shape=jax.ShapeDtypeStruct((N_DST, D), jnp.float32),
        interpret=interpret,
    )(S, S, C3, C3, wit, wot, bi, bo)


def _e_out_body(x, w, b, o):
    o[...] = jnp.dot(x[...], w[...], preferred_element_type=jnp.float32) + b[...]


def _e_out(ef, wrt, br, interpret=False):
    BE = 2000
    return pl.pallas_call(
        _e_out_body,
        grid=(E // BE,),
        in_specs=[
            pl.BlockSpec((BE, D), lambda i: (i, 0)),
            pl.BlockSpec((D, D), lambda i: (0, 0)),
            pl.BlockSpec((1, D), lambda i: (0, 0)),
        ],
        out_specs=pl.BlockSpec((BE, D), lambda i: (i, 0)),
        out_shape=jax.ShapeDtypeStruct((E, D), jnp.float32),
        interpret=interpret,
    )(ef, wrt, br)


def kernel(h_src, h_dst, edge_feat, edge_index, forward_mask, reverse_mask,
           WI, bI, WO, bO, WR, bR):
    src = edge_index[0]
    dst = edge_index[1]
    fwd = forward_mask.astype(jnp.int32)
    hcat = jnp.concatenate([h_src[:, :DH], h_src[:, DH:]], axis=0)
    src3 = src.reshape(NS, NCH, CHUNK)
    dst3 = dst.reshape(NS, NCH, CHUNK)
    fwd3 = fwd.reshape(NS, NCH, CHUNK)

    S, C = _sc_seg(hcat, edge_feat, src3, dst3, fwd3)
    C3 = C.reshape(NC, TROWS, 1)
    n_out = _n_assemble(S, C3, WI.T, WO.T,
                        bI.reshape(1, D), bO.reshape(1, D))
    e_out = _e_out(edge_feat, WR.T, bR.reshape(1, D))
    return (n_out, e_out)
